# baseline (speedup 1.0000x reference)
import functools

import jax
import jax.numpy as jnp
import numpy as np
from jax import lax
from jax.experimental import pallas as pl
from jax.experimental.pallas import tpu as pltpu
from jax.experimental.pallas import tpu_sc as plsc

SEQ_LEN = 200
VOCAB = 1000000
EMBED_DIM = 64
BATCH = 1024

NUM_CORES = 2
NUM_SUBCORES = 16
NUM_WORKERS = NUM_CORES * NUM_SUBCORES

TOTAL = BATCH * SEQ_LEN
ROWS_PER_WORKER = TOTAL // NUM_WORKERS
CHUNK = 128
CHUNKS_PER_WORKER = ROWS_PER_WORKER // CHUNK
IDX_ROWS = TOTAL // CHUNK


def _positional_encoding():
    pos = np.arange(SEQ_LEN, dtype=np.float64)[:, None]
    i = np.arange(EMBED_DIM, dtype=np.float64)[None, :]
    exponent = (i - np.mod(i, 2)) / EMBED_DIM
    angle = pos / np.power(10000.0, exponent)
    pe = np.where(np.mod(np.arange(EMBED_DIM), 2)[None, :] == 0,
                  np.sin(angle), np.cos(angle))
    return pe.astype(np.float32)


_PE = _positional_encoding()


NCHUNK = ROWS_PER_WORKER // SEQ_LEN
NBUF = 8
LAG_G = 2
LAG_O = 4
PAD_DIM = 2 * EMBED_DIM


def _sc_body(table_hbm, idx_hbm, pe_hbm, out_hbm, idx_v, pe_s, bufs, *sems):
    sid = lax.axis_index("s")
    wid = sid * NUM_CORES + lax.axis_index("c")
    row_base = wid * ROWS_PER_WORKER

    pltpu.sync_copy(idx_hbm.at[pl.ds(row_base, ROWS_PER_WORKER)], idx_v)

    @pl.when(sid == 0)
    def _():
        pltpu.sync_copy(pe_hbm, pe_s)

    plsc.subcore_barrier()

    fills, g1s, g2s, outs = {}, {}, {}, {}
    for t in range(NCHUNK + LAG_O):
        if t < NCHUNK:
            b = t % NBUF
            if t >= NBUF:
                outs[t - NBUF].wait()
            fills[t] = pltpu.async_copy(pe_s, bufs.at[b], sems[b])
        j = t - LAG_G
        if 0 <= j < NCHUNK:
            b = j % NBUF
            fills[j].wait()
            base = j * SEQ_LEN
            g1s[j] = pltpu.async_copy(
                table_hbm.at[idx_v.at[pl.ds(base, 128)]],
                bufs.at[b].at[pl.ds(0, 128)], sems[b], add=True)
            g2s[j] = pltpu.async_copy(
                table_hbm.at[idx_v.at[pl.ds(base + 128, 72)]],
                bufs.at[b].at[pl.ds(128, 72)], sems[b], add=True)
        j = t - LAG_O
        if 0 <= j < NCHUNK:
            b = j % NBUF
            g1s[j].wait()
            g2s[j].wait()
            outs[j] = pltpu.async_copy(
                bufs.at[b], out_hbm.at[wid * NCHUNK + j], sems[b])
    for j in range(NCHUNK - NBUF, NCHUNK):
        outs[j].wait()


TC_K = 16384
TC_GRID = (VOCAB + 2 * TC_K - 1) // (2 * TC_K)
PACK_ROWS = TC_K * TC_GRID
LIN_ROWS = 2 * PACK_ROWS


def _repack_body(a_ref, b_ref, out_ref):
    out_ref[...] = jnp.concatenate([a_ref[...].T, b_ref[...].T], axis=1)


def _repack(table_t):
    return pl.pallas_call(
        _repack_body,
        grid=(TC_GRID,),
        in_specs=[
            pl.BlockSpec((EMBED_DIM, TC_K), lambda i: (0, 2 * i)),
            pl.BlockSpec(
                (EMBED_DIM, TC_K),
                lambda i: (0, jnp.minimum(2 * i + 1, VOCAB // TC_K - 1)),
            ),
        ],
        out_specs=pl.BlockSpec((TC_K, PAD_DIM), lambda i: (i, 0)),
        out_shape=jax.ShapeDtypeStruct((PACK_ROWS, PAD_DIM), jnp.float32),
    )(table_t, table_t)


@jax.jit
def _embed(inputs_flat, table, pe):
    mesh = plsc.VectorSubcoreMesh(core_axis_name="c", subcore_axis_name="s")
    call = functools.partial(
        pl.kernel,
        mesh=mesh,
        out_type=jax.ShapeDtypeStruct((BATCH, SEQ_LEN, EMBED_DIM), jnp.float32),
        scratch_types=[
            pltpu.VMEM((ROWS_PER_WORKER,), jnp.int32),
            pltpu.VMEM_SHARED((SEQ_LEN, EMBED_DIM), jnp.float32),
            pltpu.VMEM((NBUF, SEQ_LEN, EMBED_DIM), jnp.float32),
        ] + [pltpu.SemaphoreType.DMA] * NBUF + [
        ],
        compiler_params=pltpu.CompilerParams(use_tc_tiling_on_sc=False),
    )(_sc_body)
    return call(table, inputs_flat, pe)


def kernel(inputs, table):
    v = inputs.reshape(TOTAL).astype(jnp.int32)
    idx = (v & ~(2 * TC_K - 1)) + 2 * (v & (TC_K - 1)) + ((v >> 14) & 1)
    t_lin = _repack(table.T).reshape(LIN_ROWS, EMBED_DIM)
    return _embed(idx, t_lin, _PE)


# speedup vs baseline: 1.0360x; 1.0360x over previous
import functools

import jax
import jax.numpy as jnp
import numpy as np
from jax import lax
from jax.experimental import pallas as pl
from jax.experimental.pallas import tpu as pltpu
from jax.experimental.pallas import tpu_sc as plsc

SEQ_LEN = 200
VOCAB = 1000000
EMBED_DIM = 64
BATCH = 1024

NUM_CORES = 2
NUM_SUBCORES = 16
NUM_WORKERS = NUM_CORES * NUM_SUBCORES

TOTAL = BATCH * SEQ_LEN
ROWS_PER_WORKER = TOTAL // NUM_WORKERS
CHUNK = 128
CHUNKS_PER_WORKER = ROWS_PER_WORKER // CHUNK
IDX_ROWS = TOTAL // CHUNK


def _positional_encoding():
    pos = np.arange(SEQ_LEN, dtype=np.float64)[:, None]
    i = np.arange(EMBED_DIM, dtype=np.float64)[None, :]
    exponent = (i - np.mod(i, 2)) / EMBED_DIM
    angle = pos / np.power(10000.0, exponent)
    pe = np.where(np.mod(np.arange(EMBED_DIM), 2)[None, :] == 0,
                  np.sin(angle), np.cos(angle))
    return pe.astype(np.float32)


_PE = _positional_encoding()


NCHUNK = ROWS_PER_WORKER // SEQ_LEN
NBUF = 8
LAG_G = 2
LAG_O = 4
PAD_DIM = 2 * EMBED_DIM


def _sc_body(table_hbm, idx_hbm, pe_hbm, out_hbm, idx_v, pe_s, bufs, *sems):
    sid = lax.axis_index("s")
    wid = sid * NUM_CORES + lax.axis_index("c")
    row_base = wid * ROWS_PER_WORKER

    pltpu.sync_copy(idx_hbm.at[pl.ds(row_base, ROWS_PER_WORKER)], idx_v)

    @pl.when(sid == 0)
    def _():
        pltpu.sync_copy(pe_hbm, pe_s)

    plsc.subcore_barrier()

    fills, g1s, g2s, outs = {}, {}, {}, {}
    for t in range(NCHUNK + LAG_O):
        if t < NCHUNK:
            b = t % NBUF
            if t >= NBUF:
                outs[t - NBUF].wait()
            fills[t] = pltpu.async_copy(pe_s, bufs.at[b], sems[b])
        j = t - LAG_G
        if 0 <= j < NCHUNK:
            b = j % NBUF
            fills[j].wait()
            base = j * SEQ_LEN
            g1s[j] = pltpu.async_copy(
                table_hbm.at[idx_v.at[pl.ds(base, 128)]],
                bufs.at[b].at[pl.ds(0, 128)], sems[b], add=True)
            g2s[j] = pltpu.async_copy(
                table_hbm.at[idx_v.at[pl.ds(base + 128, 72)]],
                bufs.at[b].at[pl.ds(128, 72)], sems[b], add=True)
        j = t - LAG_O
        if 0 <= j < NCHUNK:
            b = j % NBUF
            g1s[j].wait()
            g2s[j].wait()
            outs[j] = pltpu.async_copy(
                bufs.at[b], out_hbm.at[wid * NCHUNK + j], sems[b])
    for j in range(NCHUNK - NBUF, NCHUNK):
        outs[j].wait()


TC_K = 16384
TC_GRID = (VOCAB + 2 * TC_K - 1) // (2 * TC_K)
PACK_ROWS = TC_K * TC_GRID
LIN_ROWS = 2 * PACK_ROWS


def _repack_body(a_ref, b_ref, out_ref):
    out_ref[...] = jnp.concatenate([a_ref[...].T, b_ref[...].T], axis=1)


def _repack(table_t):
    return pl.pallas_call(
        _repack_body,
        grid=(TC_GRID,),
        in_specs=[
            pl.BlockSpec((EMBED_DIM, TC_K), lambda i: (0, 2 * i)),
            pl.BlockSpec(
                (EMBED_DIM, TC_K),
                lambda i: (0, jnp.minimum(2 * i + 1,
                                          (VOCAB + TC_K - 1) // TC_K - 1)),
            ),
        ],
        out_specs=pl.BlockSpec((TC_K, PAD_DIM), lambda i: (i, 0)),
        out_shape=jax.ShapeDtypeStruct((PACK_ROWS, PAD_DIM), jnp.float32),
        compiler_params=pltpu.CompilerParams(
            vmem_limit_bytes=56 * 1024 * 1024),
    )(table_t, table_t)


@jax.jit
def _embed(inputs_flat, table, pe):
    mesh = plsc.VectorSubcoreMesh(core_axis_name="c", subcore_axis_name="s")
    call = functools.partial(
        pl.kernel,
        mesh=mesh,
        out_type=jax.ShapeDtypeStruct((BATCH, SEQ_LEN, EMBED_DIM), jnp.float32),
        scratch_types=[
            pltpu.VMEM((ROWS_PER_WORKER,), jnp.int32),
            pltpu.VMEM_SHARED((SEQ_LEN, EMBED_DIM), jnp.float32),
            pltpu.VMEM((NBUF, SEQ_LEN, EMBED_DIM), jnp.float32),
        ] + [pltpu.SemaphoreType.DMA] * NBUF + [
        ],
        compiler_params=pltpu.CompilerParams(use_tc_tiling_on_sc=False),
    )(_sc_body)
    return call(table, inputs_flat, pe)


def kernel(inputs, table):
    v = inputs.reshape(TOTAL).astype(jnp.int32)
    idx = (v & ~(2 * TC_K - 1)) + 2 * (v & (TC_K - 1)) + ((v >> 14) & 1)
    t_lin = _repack(table.T).reshape(LIN_ROWS, EMBED_DIM)
    return _embed(idx, t_lin, _PE)


# speedup vs baseline: 1.0420x; 1.0058x over previous
import functools

import jax
import jax.numpy as jnp
import numpy as np
from jax import lax
from jax.experimental import pallas as pl
from jax.experimental.pallas import tpu as pltpu
from jax.experimental.pallas import tpu_sc as plsc

SEQ_LEN = 200
VOCAB = 1000000
EMBED_DIM = 64
BATCH = 1024

NUM_CORES = 2
NUM_SUBCORES = 16
NUM_WORKERS = NUM_CORES * NUM_SUBCORES

TOTAL = BATCH * SEQ_LEN
ROWS_PER_WORKER = TOTAL // NUM_WORKERS
CHUNK = 128
CHUNKS_PER_WORKER = ROWS_PER_WORKER // CHUNK
IDX_ROWS = TOTAL // CHUNK


def _positional_encoding():
    pos = np.arange(SEQ_LEN, dtype=np.float64)[:, None]
    i = np.arange(EMBED_DIM, dtype=np.float64)[None, :]
    exponent = (i - np.mod(i, 2)) / EMBED_DIM
    angle = pos / np.power(10000.0, exponent)
    pe = np.where(np.mod(np.arange(EMBED_DIM), 2)[None, :] == 0,
                  np.sin(angle), np.cos(angle))
    return pe.astype(np.float32)


_PE = _positional_encoding()


NCHUNK = ROWS_PER_WORKER // SEQ_LEN
NBUF = 8
LAG_G = 3
LAG_O = 6
PAD_DIM = 2 * EMBED_DIM


def _sc_body(table_hbm, idx_hbm, pe_hbm, out_hbm, idx_v, pe_s, bufs,
             isem, *sems):
    sid = lax.axis_index("s")
    wid = sid * NUM_CORES + lax.axis_index("c")
    row_base = wid * ROWS_PER_WORKER

    idx_cp = pltpu.async_copy(
        idx_hbm.at[pl.ds(row_base, ROWS_PER_WORKER)], idx_v, isem)

    @pl.when(sid == 0)
    def _():
        pltpu.sync_copy(pe_hbm, pe_s)

    plsc.subcore_barrier()

    fills, g1s, g2s, outs = {}, {}, {}, {}
    for t in range(NCHUNK + LAG_O):
        if t < NCHUNK:
            b = t % NBUF
            if t >= NBUF:
                outs[t - NBUF].wait()
            fills[t] = pltpu.async_copy(pe_s, bufs.at[b], sems[b])
        j = t - LAG_G
        if 0 <= j < NCHUNK:
            if j == 0:
                idx_cp.wait()
            b = j % NBUF
            fills[j].wait()
            base = j * SEQ_LEN
            g1s[j] = pltpu.async_copy(
                table_hbm.at[idx_v.at[pl.ds(base, 128)]],
                bufs.at[b].at[pl.ds(0, 128)], sems[b], add=True)
            g2s[j] = pltpu.async_copy(
                table_hbm.at[idx_v.at[pl.ds(base + 128, 72)]],
                bufs.at[b].at[pl.ds(128, 72)], sems[b], add=True)
        j = t - LAG_O
        if 0 <= j < NCHUNK:
            b = j % NBUF
            g1s[j].wait()
            g2s[j].wait()
            outs[j] = pltpu.async_copy(
                bufs.at[b], out_hbm.at[wid * NCHUNK + j], sems[b])
    for j in range(NCHUNK - NBUF, NCHUNK):
        outs[j].wait()


TC_K = 16384
TC_GRID = (VOCAB + 2 * TC_K - 1) // (2 * TC_K)
PACK_ROWS = TC_K * TC_GRID
LIN_ROWS = 2 * PACK_ROWS


def _repack_body(a_ref, b_ref, out_ref):
    out_ref[...] = jnp.concatenate([a_ref[...].T, b_ref[...].T], axis=1)


def _repack(table_t):
    return pl.pallas_call(
        _repack_body,
        grid=(TC_GRID,),
        in_specs=[
            pl.BlockSpec((EMBED_DIM, TC_K), lambda i: (0, 2 * i)),
            pl.BlockSpec(
                (EMBED_DIM, TC_K),
                lambda i: (0, jnp.minimum(2 * i + 1,
                                          (VOCAB + TC_K - 1) // TC_K - 1)),
            ),
        ],
        out_specs=pl.BlockSpec((TC_K, PAD_DIM), lambda i: (i, 0)),
        out_shape=jax.ShapeDtypeStruct((PACK_ROWS, PAD_DIM), jnp.float32),
        compiler_params=pltpu.CompilerParams(
            vmem_limit_bytes=56 * 1024 * 1024),
    )(table_t, table_t)


@jax.jit
def _embed(inputs_flat, table, pe):
    mesh = plsc.VectorSubcoreMesh(core_axis_name="c", subcore_axis_name="s")
    call = functools.partial(
        pl.kernel,
        mesh=mesh,
        out_type=jax.ShapeDtypeStruct((BATCH, SEQ_LEN, EMBED_DIM), jnp.float32),
        scratch_types=[
            pltpu.VMEM((ROWS_PER_WORKER,), jnp.int32),
            pltpu.VMEM_SHARED((SEQ_LEN, EMBED_DIM), jnp.float32),
            pltpu.VMEM((NBUF, SEQ_LEN, EMBED_DIM), jnp.float32),
        ] + [pltpu.SemaphoreType.DMA] * (NBUF + 1) + [
        ],
        compiler_params=pltpu.CompilerParams(use_tc_tiling_on_sc=False),
    )(_sc_body)
    return call(table, inputs_flat, pe)


def kernel(inputs, table):
    v = inputs.reshape(TOTAL).astype(jnp.int32)
    idx = (v & ~(2 * TC_K - 1)) + 2 * (v & (TC_K - 1)) + ((v >> 14) & 1)
    t_lin = _repack(table.T).reshape(LIN_ROWS, EMBED_DIM)
    return _embed(idx, t_lin, _PE)
